# baseline (speedup 1.0000x reference)
import functools

import jax
import jax.numpy as jnp
from jax.experimental import pallas as pl
from jax.experimental.pallas import tpu as pltpu


_ROWS_PER_BLOCK = 2048


def _vq_block_kernel(x_ref, w1_ref, b1_ref, w2_ref, b2_ref, cb_ref, csq_ref,
                     idx_ref, loss_ref, d2_scr, esq_scr):
    i = pl.program_id(0)
    n_blocks = pl.num_programs(0) - 1

    @pl.when(i == 0)
    def _init():
        loss_ref[...] = jnp.zeros((1, 1), jnp.float32)

    d2_prev = esq_scr[...] + d2_scr[...]
    dist_prev = jnp.sqrt(jnp.maximum(d2_prev, 0.0))
    idx_ref[...] = jnp.argmin(dist_prev, axis=1,
                              keepdims=True).astype(jnp.int32)
    m = jnp.min(d2_prev, axis=1, keepdims=True)
    tail_sum = jnp.sum(jnp.maximum(m, 0.0)).reshape(1, 1)
    loss_ref[...] += jnp.where(i > 0, tail_sum, 0.0)

    @pl.when(i < n_blocks)
    def _mm():
        x = x_ref[...]
        h = jnp.dot(x, w1_ref[...], preferred_element_type=jnp.float32)
        h = jnp.maximum(h + b1_ref[...], 0.0)
        e = jnp.dot(h, w2_ref[...], preferred_element_type=jnp.float32)
        e = e + b2_ref[...]
        xc = jnp.dot(e, cb_ref[...].T, preferred_element_type=jnp.float32)
        d2_scr[...] = csq_ref[...] - 2.0 * xc
        esq_scr[...] = jnp.sum(e * e, axis=1, keepdims=True)


@functools.partial(jax.jit, static_argnames=())
def kernel(x, W1, b1, W2, b2, codebook, Wd, bd):
    B, T, D = x.shape
    N = B * T
    R = _ROWS_PER_BLOCK
    xf = x.reshape(N, D)
    csq = jnp.sum(codebook * codebook, axis=1)[None, :]
    n_blocks = N // R

    idx_col, loss_sum = pl.pallas_call(
        _vq_block_kernel,
        grid=(n_blocks + 1,),
        in_specs=[
            pl.BlockSpec((R, D), lambda i: (jnp.minimum(i, n_blocks - 1), 0)),
            pl.BlockSpec(W1.shape, lambda i: (0, 0)),
            pl.BlockSpec((1, b1.shape[0]), lambda i: (0, 0)),
            pl.BlockSpec(W2.shape, lambda i: (0, 0)),
            pl.BlockSpec((1, b2.shape[0]), lambda i: (0, 0)),
            pl.BlockSpec(codebook.shape, lambda i: (0, 0)),
            pl.BlockSpec((1, codebook.shape[0]), lambda i: (0, 0)),
        ],
        out_specs=[
            pl.BlockSpec((R, 1), lambda i: (jnp.maximum(i - 1, 0), 0)),
            pl.BlockSpec((1, 1), lambda i: (0, 0)),
        ],
        out_shape=[
            jax.ShapeDtypeStruct((N, 1), jnp.int32),
            jax.ShapeDtypeStruct((1, 1), jnp.float32),
        ],
        scratch_shapes=[
            pltpu.VMEM((R, 128), jnp.float32),
            pltpu.VMEM((R, 1), jnp.float32),
        ],
    )(xf, W1, b1[None, :], W2, b2[None, :], codebook, csq)

    token_indices = idx_col.reshape(B, T)
    loss = loss_sum[0, 0] / jnp.float32(N * codebook.shape[1])
    commitment_loss = loss
    codebook_loss = loss
    total_loss = commitment_loss + 0.25 * codebook_loss
    return (token_indices, commitment_loss, codebook_loss, total_loss)


# speedup vs baseline: 1.0858x; 1.0858x over previous
import functools

import jax
import jax.numpy as jnp
from jax.experimental import pallas as pl


_ROWS_PER_BLOCK = 2048


def _vq_block_kernel(x_ref, w1_ref, b1_ref, w2_ref, b2_ref, cb_ref, csq_ref,
                     idx_ref, loss_ref):
    i = pl.program_id(0)
    half = _ROWS_PER_BLOCK // 2

    def _sub(lo):
        x = x_ref[pl.ds(lo, half), :]
        h = jnp.dot(x, w1_ref[...], preferred_element_type=jnp.float32)
        h = jnp.maximum(h + b1_ref[...], 0.0)
        e = jnp.dot(h, w2_ref[...], preferred_element_type=jnp.float32)
        e = e + b2_ref[...]
        xc = jnp.dot(e, cb_ref[...].T, preferred_element_type=jnp.float32)
        esq = jnp.sum(e * e, axis=1, keepdims=True)
        d2 = esq + csq_ref[...] - 2.0 * xc
        dist = jnp.sqrt(jnp.maximum(d2, 0.0))
        idx_ref[pl.ds(lo, half), :] = jnp.argmin(
            dist, axis=1, keepdims=True).astype(jnp.int32)
        m = jnp.min(d2, axis=1, keepdims=True)
        return jnp.sum(jnp.maximum(m, 0.0))

    s0 = _sub(0)
    s1 = _sub(half)
    block_sum = (s0 + s1).reshape(1, 1)

    @pl.when(i == 0)
    def _init():
        loss_ref[...] = jnp.zeros((1, 1), jnp.float32)

    loss_ref[...] += block_sum


@functools.partial(jax.jit, static_argnames=())
def kernel(x, W1, b1, W2, b2, codebook, Wd, bd):
    B, T, D = x.shape
    N = B * T
    R = _ROWS_PER_BLOCK
    xf = x.reshape(N, D)
    csq = jnp.sum(codebook * codebook, axis=1)[None, :]
    grid = N // R

    idx_col, loss_sum = pl.pallas_call(
        _vq_block_kernel,
        grid=(grid,),
        in_specs=[
            pl.BlockSpec((R, D), lambda i: (i, 0)),
            pl.BlockSpec(W1.shape, lambda i: (0, 0)),
            pl.BlockSpec((1, b1.shape[0]), lambda i: (0, 0)),
            pl.BlockSpec(W2.shape, lambda i: (0, 0)),
            pl.BlockSpec((1, b2.shape[0]), lambda i: (0, 0)),
            pl.BlockSpec(codebook.shape, lambda i: (0, 0)),
            pl.BlockSpec((1, codebook.shape[0]), lambda i: (0, 0)),
        ],
        out_specs=[
            pl.BlockSpec((R, 1), lambda i: (i, 0)),
            pl.BlockSpec((1, 1), lambda i: (0, 0)),
        ],
        out_shape=[
            jax.ShapeDtypeStruct((N, 1), jnp.int32),
            jax.ShapeDtypeStruct((1, 1), jnp.float32),
        ],
    )(xf, W1, b1[None, :], W2, b2[None, :], codebook, csq)

    token_indices = idx_col.reshape(B, T)
    loss = loss_sum[0, 0] / jnp.float32(N * codebook.shape[1])
    commitment_loss = loss
    codebook_loss = loss
    total_loss = commitment_loss + 0.25 * codebook_loss
    return (token_indices, commitment_loss, codebook_loss, total_loss)


# speedup vs baseline: 1.1073x; 1.0198x over previous
import functools

import jax
import jax.numpy as jnp
from jax.experimental import pallas as pl


_ROWS_PER_BLOCK = 2048


def _vq_block_kernel(x_ref, w1_ref, b1_ref, w2_ref, b2_ref, cb_ref, csq_ref,
                     idx_ref, loss_ref):
    i = pl.program_id(0)
    x = x_ref[...]
    h = jnp.dot(x, w1_ref[...], preferred_element_type=jnp.float32)
    h = jnp.maximum(h + b1_ref[...], 0.0)
    e = jnp.dot(h, w2_ref[...], preferred_element_type=jnp.float32)
    e = e + b2_ref[...]
    xc = jnp.dot(e, cb_ref[...].T, preferred_element_type=jnp.float32)
    esq = jnp.sum(e * e, axis=1, keepdims=True)
    d2 = esq + csq_ref[...] - 2.0 * xc
    dist = jnp.sqrt(jnp.maximum(d2, 0.0))
    idx_ref[...] = jnp.argmin(dist, axis=1, keepdims=True).astype(jnp.int32)
    m = jnp.min(d2, axis=1, keepdims=True)
    block_sum = jnp.sum(jnp.maximum(m, 0.0)).reshape(1, 1)

    @pl.when(i == 0)
    def _init():
        loss_ref[...] = jnp.zeros((1, 1), jnp.float32)

    loss_ref[...] += block_sum


@functools.partial(jax.jit, static_argnames=())
def kernel(x, W1, b1, W2, b2, codebook, Wd, bd):
    B, T, D = x.shape
    N = B * T
    R = _ROWS_PER_BLOCK
    xf = x.reshape(N, D)
    csq = jnp.sum(codebook * codebook, axis=1)[None, :]
    grid = N // R

    idx_col, loss_sum = pl.pallas_call(
        _vq_block_kernel,
        grid=(grid,),
        in_specs=[
            pl.BlockSpec((R, D), lambda i: (i, 0)),
            pl.BlockSpec(W1.shape, lambda i: (0, 0)),
            pl.BlockSpec((1, b1.shape[0]), lambda i: (0, 0)),
            pl.BlockSpec(W2.shape, lambda i: (0, 0)),
            pl.BlockSpec((1, b2.shape[0]), lambda i: (0, 0)),
            pl.BlockSpec(codebook.shape, lambda i: (0, 0)),
            pl.BlockSpec((1, codebook.shape[0]), lambda i: (0, 0)),
        ],
        out_specs=[
            pl.BlockSpec((R, 1), lambda i: (i, 0)),
            pl.BlockSpec((1, 1), lambda i: (0, 0)),
        ],
        out_shape=[
            jax.ShapeDtypeStruct((N, 1), jnp.int32),
            jax.ShapeDtypeStruct((1, 1), jnp.float32),
        ],
    )(xf, W1, b1[None, :], W2, b2[None, :], codebook, csq)

    token_indices = idx_col.reshape(B, T)
    loss = loss_sum[0, 0] / jnp.float32(N * codebook.shape[1])
    commitment_loss = loss
    codebook_loss = loss
    total_loss = commitment_loss + 0.25 * codebook_loss
    return (token_indices, commitment_loss, codebook_loss, total_loss)
